# baseline (speedup 1.0000x reference)
import jax
import jax.numpy as jnp
from jax import lax
from jax.experimental import pallas as pl
from jax.experimental.pallas import tpu as pltpu
from jax.experimental.pallas import tpu_sc as plsc

N = 10000
E = 320000
INC = 128
HID = 32
NHH = 4096

NC, NS, L = 2, 16, 16
NW = NC * NS
CH = 128
NCHUNK = E // CH
CW_Q, CW_R = divmod(NCHUNK, NW)
CPW = 80
NBUF = 8
KAHEAD = 4
N_PAD = 10112
RPT = N_PAD // NS
W1 = HID + L


def _proj_body(x_ref, wl_ref, wr_ref, p_ref, r_ref):
    x = x_ref[...]
    ones = jnp.ones((x.shape[0], L), jnp.float32)
    p_ref[...] = jnp.concatenate(
        [jnp.dot(x, wl_ref[...], preferred_element_type=jnp.float32), ones],
        axis=1)
    r_ref[...] = jnp.dot(x, wr_ref[...], preferred_element_type=jnp.float32)


def _proj(x, Wl, Wr, br):
    n, d = x.shape
    return pl.pallas_call(
        _proj_body,
        grid=(n // br,),
        in_specs=[
            pl.BlockSpec((br, d), lambda i: (i, 0)),
            pl.BlockSpec(Wl.shape, lambda i: (0, 0)),
            pl.BlockSpec(Wr.shape, lambda i: (0, 0)),
        ],
        out_specs=[
            pl.BlockSpec((br, W1), lambda i: (i, 0)),
            pl.BlockSpec((br, HID), lambda i: (i, 0)),
        ],
        out_shape=[jax.ShapeDtypeStruct((n, W1), jnp.float32),
                   jax.ShapeDtypeStruct((n, HID), jnp.float32)],
    )(x, Wl, Wr)


def _mid_body(a_ref, r_ref, b_ref, wl_ref, wr_ref, p_ref, r2_ref):
    a0 = a_ref[0]
    a1 = a_ref[1]
    deg = jnp.maximum(a0[:, HID:HID + 1] + a1[:, HID:HID + 1], 1.0)
    h = jnp.maximum(
        (a0[:, :HID] + a1[:, :HID]) / deg + r_ref[...] + b_ref[0:1, :], 0.0)
    p_ref[...] = jnp.dot(h, wl_ref[...], preferred_element_type=jnp.float32)
    r2_ref[...] = jnp.dot(h, wr_ref[...], preferred_element_type=jnp.float32)


def _mid(agg1, r1, b1, Wl2, Wr2, br):
    n = r1.shape[0]
    row = lambda i: (i, 0)
    whole = lambda i: (0, 0)
    return pl.pallas_call(
        _mid_body,
        grid=(n // br,),
        in_specs=[
            pl.BlockSpec((NC, br, W1), lambda i: (0, i, 0)),
            pl.BlockSpec((br, HID), row),
            pl.BlockSpec((8, HID), whole),
            pl.BlockSpec((HID, HID), whole),
            pl.BlockSpec((HID, HID), whole),
        ],
        out_specs=[
            pl.BlockSpec((br, HID), row),
            pl.BlockSpec((br, HID), row),
        ],
        out_shape=[jax.ShapeDtypeStruct((n, HID), jnp.float32)] * 2,
    )(agg1, r1, b1, Wl2, Wr2)


def _head_body(a_ref, da_ref, r_ref, b_ref, wfc_ref, bfc_ref, o_ref):
    deg = jnp.maximum(
        da_ref[0, :, HID:HID + 1] + da_ref[1, :, HID:HID + 1], 1.0)
    h = jnp.maximum(
        (a_ref[0] + a_ref[1]) / deg + r_ref[...] + b_ref[0:1, :], 0.0)
    o_ref[...] = (jnp.dot(h, wfc_ref[...], preferred_element_type=jnp.float32)
                  + bfc_ref[0:1, :])


def _head(agg2, agg1, r2, b2, Wfc, bfc, br):
    n = r2.shape[0]
    row = lambda i: (i, 0)
    whole = lambda i: (0, 0)
    return pl.pallas_call(
        _head_body,
        grid=(n // br,),
        in_specs=[
            pl.BlockSpec((NC, br, HID), lambda i: (0, i, 0)),
            pl.BlockSpec((NC, br, W1), lambda i: (0, i, 0)),
            pl.BlockSpec((br, HID), row),
            pl.BlockSpec((8, HID), whole),
            pl.BlockSpec((HID, NHH), whole),
            pl.BlockSpec((8, NHH), whole),
        ],
        out_specs=pl.BlockSpec((br, NHH), row),
        out_shape=jax.ShapeDtypeStruct((n, NHH), jnp.float32),
    )(agg2, agg1, r2, b2, Wfc, bfc)


def _agg_impl(width, p_hbm, eidx_hbm, za_hbm, agg_out,
              src_v, dst_v, rows_v, agg_sh, tab_sh, gsems, ssems):
    c = lax.axis_index("c")
    s = lax.axis_index("s")
    wid = s * NC + c
    cw = CW_Q + jnp.where(wid < CW_R, 1, 0)
    start = CW_Q * wid + jnp.minimum(wid, CW_R)

    pltpu.sync_copy(za_hbm.at[pl.ds(s * RPT, RPT)],
                    agg_sh.at[pl.ds(s * RPT, RPT)])

    tpr = N // NS
    pltpu.sync_copy(p_hbm.at[pl.ds(s * tpr, tpr)],
                    tab_sh.at[pl.ds(s * tpr, tpr)])
    pltpu.sync_copy(eidx_hbm.at[0, pl.ds(start, CW_Q)],
                    src_v.at[pl.ds(0, CW_Q)])
    pltpu.sync_copy(eidx_hbm.at[1, pl.ds(start, CW_Q)],
                    dst_v.at[pl.ds(0, CW_Q)])

    @pl.when(cw > CW_Q)
    def _tail():
        pltpu.sync_copy(eidx_hbm.at[0, pl.ds(start + CW_Q, 1)],
                        src_v.at[pl.ds(CW_Q, 1)])
        pltpu.sync_copy(eidx_hbm.at[1, pl.ds(start + CW_Q, 1)],
                        dst_v.at[pl.ds(CW_Q, 1)])

    plsc.subcore_barrier()

    for bb in range(KAHEAD):
        pltpu.async_copy(tab_sh.at[src_v.at[bb]], rows_v.at[bb], gsems[bb])

    def outer(g, carry):
        for bb in range(NBUF):
            j = g * NBUF + bb
            bn = (bb + KAHEAD) % NBUF

            @pl.when((j + KAHEAD >= NBUF) & (j + KAHEAD < CPW)
                     & (j + KAHEAD - NBUF < cw))
            def _free():
                pltpu.make_async_copy(
                    rows_v.at[bn],
                    agg_sh.at[dst_v.at[j + KAHEAD - NBUF]],
                    ssems[bn]).wait()

            @pl.when(j + KAHEAD < cw)
            def _prefetch():
                pltpu.async_copy(
                    tab_sh.at[src_v.at[j + KAHEAD]], rows_v.at[bn], gsems[bn])

            @pl.when(j < cw)
            def _work():
                pltpu.make_async_copy(
                    tab_sh.at[src_v.at[j]], rows_v.at[bb], gsems[bb]).wait()
                pltpu.async_copy(
                    rows_v.at[bb], agg_sh.at[dst_v.at[j]], ssems[bb], add=True)
        return carry

    lax.fori_loop(0, CPW // NBUF, outer, 0)
    for bb in range(NBUF):
        j = CPW - NBUF + bb

        @pl.when(j < cw)
        def _drain():
            pltpu.make_async_copy(
                rows_v.at[bb], agg_sh.at[dst_v.at[j]], ssems[bb]).wait()

    plsc.subcore_barrier()
    pltpu.sync_copy(agg_sh.at[pl.ds(s * RPT, RPT)],
                    agg_out.at[c, pl.ds(s * RPT, RPT)])


def _agg1_body(p_hbm, eidx_hbm, za_hbm, agg_out,
               src_v, dst_v, rows_v, agg_sh, tab_sh, *sems):
    _agg_impl(W1, p_hbm, eidx_hbm, za_hbm, agg_out,
              src_v, dst_v, rows_v, agg_sh, tab_sh, sems[:NBUF], sems[NBUF:])


def _agg2_body(p_hbm, eidx_hbm, za_hbm, agg_out,
               src_v, dst_v, rows_v, agg_sh, tab_sh, *sems):
    _agg_impl(HID, p_hbm, eidx_hbm, za_hbm, agg_out,
              src_v, dst_v, rows_v, agg_sh, tab_sh, sems[:NBUF], sems[NBUF:])


def _edge_agg(p, eidx, za, width, body):
    mesh = plsc.VectorSubcoreMesh(core_axis_name="c", subcore_axis_name="s")
    f = pl.kernel(
        body,
        out_type=jax.ShapeDtypeStruct((NC, N_PAD, width), jnp.float32),
        mesh=mesh,
        scratch_types=[
            pltpu.VMEM((CPW, CH), jnp.int32),
            pltpu.VMEM((CPW, CH), jnp.int32),
            pltpu.VMEM((NBUF, CH, width), jnp.float32),
            pltpu.VMEM_SHARED((N_PAD, width), jnp.float32),
            pltpu.VMEM_SHARED((N, width), jnp.float32),
        ] + [pltpu.SemaphoreType.DMA] * (2 * NBUF),
        compiler_params=pltpu.CompilerParams(use_tc_tiling_on_sc=False),
    )
    return f(p, eidx, za)


def kernel(x, edge_index, Wl1, Wr1, b1, Wl2, Wr2, b2, Wfc, bfc):
    eidx = edge_index.astype(jnp.int32).reshape(2, NCHUNK, CH)
    za1 = jnp.zeros((N_PAD, W1), jnp.float32)
    za2 = jnp.zeros((N_PAD, HID), jnp.float32)
    b1t = jnp.broadcast_to(b1[None, :], (8, HID))
    b2t = jnp.broadcast_to(b2[None, :], (8, HID))
    bfct = jnp.broadcast_to(bfc[None, :], (8, NHH))

    p1, r1 = _proj(x, Wl1, Wr1, 2000)
    agg1 = _edge_agg(p1, eidx, za1, W1, _agg1_body)
    p2, r2 = _mid(agg1, r1, b1t, Wl2, Wr2, 2000)
    agg2 = _edge_agg(p2, eidx, za2, HID, _agg2_body)
    out = _head(agg2, agg1, r2, b2t, Wfc, bfct, 1000)
    return out


# speedup vs baseline: 1.0021x; 1.0021x over previous
import jax
import jax.numpy as jnp
from jax import lax
from jax.experimental import pallas as pl
from jax.experimental.pallas import tpu as pltpu
from jax.experimental.pallas import tpu_sc as plsc

N = 10000
E = 320000
INC = 128
HID = 32
NHH = 4096

NC, NS, L = 2, 16, 16
NW = NC * NS
CH = 128
NCHUNK = E // CH
CW_Q, CW_R = divmod(NCHUNK, NW)
CPW = 80
NBUF = 8
KAHEAD = 6
N_PAD = 10112
RPT = N_PAD // NS
W1 = HID + L


def _proj_body(x_ref, wl_ref, wr_ref, p_ref, r_ref):
    x = x_ref[...]
    ones = jnp.ones((x.shape[0], L), jnp.float32)
    p_ref[...] = jnp.concatenate(
        [jnp.dot(x, wl_ref[...], preferred_element_type=jnp.float32), ones],
        axis=1)
    r_ref[...] = jnp.dot(x, wr_ref[...], preferred_element_type=jnp.float32)


def _proj(x, Wl, Wr, br):
    n, d = x.shape
    return pl.pallas_call(
        _proj_body,
        grid=(n // br,),
        in_specs=[
            pl.BlockSpec((br, d), lambda i: (i, 0)),
            pl.BlockSpec(Wl.shape, lambda i: (0, 0)),
            pl.BlockSpec(Wr.shape, lambda i: (0, 0)),
        ],
        out_specs=[
            pl.BlockSpec((br, W1), lambda i: (i, 0)),
            pl.BlockSpec((br, HID), lambda i: (i, 0)),
        ],
        out_shape=[jax.ShapeDtypeStruct((n, W1), jnp.float32),
                   jax.ShapeDtypeStruct((n, HID), jnp.float32)],
    )(x, Wl, Wr)


def _mid_body(a_ref, r_ref, b_ref, wl_ref, wr_ref, p_ref, r2_ref):
    a0 = a_ref[0]
    a1 = a_ref[1]
    deg = jnp.maximum(a0[:, HID:HID + 1] + a1[:, HID:HID + 1], 1.0)
    h = jnp.maximum(
        (a0[:, :HID] + a1[:, :HID]) / deg + r_ref[...] + b_ref[0:1, :], 0.0)
    p_ref[...] = jnp.dot(h, wl_ref[...], preferred_element_type=jnp.float32)
    r2_ref[...] = jnp.dot(h, wr_ref[...], preferred_element_type=jnp.float32)


def _mid(agg1, r1, b1, Wl2, Wr2, br):
    n = r1.shape[0]
    row = lambda i: (i, 0)
    whole = lambda i: (0, 0)
    return pl.pallas_call(
        _mid_body,
        grid=(n // br,),
        in_specs=[
            pl.BlockSpec((NC, br, W1), lambda i: (0, i, 0)),
            pl.BlockSpec((br, HID), row),
            pl.BlockSpec((8, HID), whole),
            pl.BlockSpec((HID, HID), whole),
            pl.BlockSpec((HID, HID), whole),
        ],
        out_specs=[
            pl.BlockSpec((br, HID), row),
            pl.BlockSpec((br, HID), row),
        ],
        out_shape=[jax.ShapeDtypeStruct((n, HID), jnp.float32)] * 2,
    )(agg1, r1, b1, Wl2, Wr2)


def _head_body(a_ref, da_ref, r_ref, b_ref, wfc_ref, bfc_ref, o_ref):
    deg = jnp.maximum(
        da_ref[0, :, HID:HID + 1] + da_ref[1, :, HID:HID + 1], 1.0)
    h = jnp.maximum(
        (a_ref[0] + a_ref[1]) / deg + r_ref[...] + b_ref[0:1, :], 0.0)
    o_ref[...] = (jnp.dot(h, wfc_ref[...], preferred_element_type=jnp.float32)
                  + bfc_ref[0:1, :])


def _head(agg2, agg1, r2, b2, Wfc, bfc, br):
    n = r2.shape[0]
    row = lambda i: (i, 0)
    whole = lambda i: (0, 0)
    return pl.pallas_call(
        _head_body,
        grid=(n // br,),
        in_specs=[
            pl.BlockSpec((NC, br, HID), lambda i: (0, i, 0)),
            pl.BlockSpec((NC, br, W1), lambda i: (0, i, 0)),
            pl.BlockSpec((br, HID), row),
            pl.BlockSpec((8, HID), whole),
            pl.BlockSpec((HID, NHH), whole),
            pl.BlockSpec((8, NHH), whole),
        ],
        out_specs=pl.BlockSpec((br, NHH), row),
        out_shape=jax.ShapeDtypeStruct((n, NHH), jnp.float32),
    )(agg2, agg1, r2, b2, Wfc, bfc)


def _agg_impl(width, p_hbm, eidx_hbm, za_hbm, agg_out,
              src_v, dst_v, rows_v, agg_sh, tab_sh, gsems, ssems):
    c = lax.axis_index("c")
    s = lax.axis_index("s")
    wid = s * NC + c
    cw = CW_Q + jnp.where(wid < CW_R, 1, 0)
    start = CW_Q * wid + jnp.minimum(wid, CW_R)

    pltpu.sync_copy(za_hbm.at[pl.ds(s * RPT, RPT)],
                    agg_sh.at[pl.ds(s * RPT, RPT)])

    tpr = N // NS
    pltpu.sync_copy(p_hbm.at[pl.ds(s * tpr, tpr)],
                    tab_sh.at[pl.ds(s * tpr, tpr)])
    pltpu.sync_copy(eidx_hbm.at[0, pl.ds(start, CW_Q)],
                    src_v.at[pl.ds(0, CW_Q)])
    pltpu.sync_copy(eidx_hbm.at[1, pl.ds(start, CW_Q)],
                    dst_v.at[pl.ds(0, CW_Q)])

    @pl.when(cw > CW_Q)
    def _tail():
        pltpu.sync_copy(eidx_hbm.at[0, pl.ds(start + CW_Q, 1)],
                        src_v.at[pl.ds(CW_Q, 1)])
        pltpu.sync_copy(eidx_hbm.at[1, pl.ds(start + CW_Q, 1)],
                        dst_v.at[pl.ds(CW_Q, 1)])

    plsc.subcore_barrier()

    for bb in range(KAHEAD):
        pltpu.async_copy(tab_sh.at[src_v.at[bb]], rows_v.at[bb], gsems[bb])

    def outer(g, carry):
        for bb in range(NBUF):
            j = g * NBUF + bb
            bn = (bb + KAHEAD) % NBUF

            @pl.when((j + KAHEAD >= NBUF) & (j + KAHEAD < CPW)
                     & (j + KAHEAD - NBUF < cw))
            def _free():
                pltpu.make_async_copy(
                    rows_v.at[bn],
                    agg_sh.at[dst_v.at[j + KAHEAD - NBUF]],
                    ssems[bn]).wait()

            @pl.when(j + KAHEAD < cw)
            def _prefetch():
                pltpu.async_copy(
                    tab_sh.at[src_v.at[j + KAHEAD]], rows_v.at[bn], gsems[bn])

            @pl.when(j < cw)
            def _work():
                pltpu.make_async_copy(
                    tab_sh.at[src_v.at[j]], rows_v.at[bb], gsems[bb]).wait()
                pltpu.async_copy(
                    rows_v.at[bb], agg_sh.at[dst_v.at[j]], ssems[bb], add=True)
        return carry

    lax.fori_loop(0, CPW // NBUF, outer, 0)
    for bb in range(NBUF):
        j = CPW - NBUF + bb

        @pl.when(j < cw)
        def _drain():
            pltpu.make_async_copy(
                rows_v.at[bb], agg_sh.at[dst_v.at[j]], ssems[bb]).wait()

    plsc.subcore_barrier()
    pltpu.sync_copy(agg_sh.at[pl.ds(s * RPT, RPT)],
                    agg_out.at[c, pl.ds(s * RPT, RPT)])


def _agg1_body(p_hbm, eidx_hbm, za_hbm, agg_out,
               src_v, dst_v, rows_v, agg_sh, tab_sh, *sems):
    _agg_impl(W1, p_hbm, eidx_hbm, za_hbm, agg_out,
              src_v, dst_v, rows_v, agg_sh, tab_sh, sems[:NBUF], sems[NBUF:])


def _agg2_body(p_hbm, eidx_hbm, za_hbm, agg_out,
               src_v, dst_v, rows_v, agg_sh, tab_sh, *sems):
    _agg_impl(HID, p_hbm, eidx_hbm, za_hbm, agg_out,
              src_v, dst_v, rows_v, agg_sh, tab_sh, sems[:NBUF], sems[NBUF:])


def _edge_agg(p, eidx, za, width, body):
    mesh = plsc.VectorSubcoreMesh(core_axis_name="c", subcore_axis_name="s")
    f = pl.kernel(
        body,
        out_type=jax.ShapeDtypeStruct((NC, N_PAD, width), jnp.float32),
        mesh=mesh,
        scratch_types=[
            pltpu.VMEM((CPW, CH), jnp.int32),
            pltpu.VMEM((CPW, CH), jnp.int32),
            pltpu.VMEM((NBUF, CH, width), jnp.float32),
            pltpu.VMEM_SHARED((N_PAD, width), jnp.float32),
            pltpu.VMEM_SHARED((N, width), jnp.float32),
        ] + [pltpu.SemaphoreType.DMA] * (2 * NBUF),
        compiler_params=pltpu.CompilerParams(use_tc_tiling_on_sc=False),
    )
    return f(p, eidx, za)


def kernel(x, edge_index, Wl1, Wr1, b1, Wl2, Wr2, b2, Wfc, bfc):
    eidx = edge_index.astype(jnp.int32).reshape(2, NCHUNK, CH)
    za1 = jnp.zeros((N_PAD, W1), jnp.float32)
    za2 = jnp.zeros((N_PAD, HID), jnp.float32)
    b1t = jnp.broadcast_to(b1[None, :], (8, HID))
    b2t = jnp.broadcast_to(b2[None, :], (8, HID))
    bfct = jnp.broadcast_to(bfc[None, :], (8, NHH))

    p1, r1 = _proj(x, Wl1, Wr1, 2000)
    agg1 = _edge_agg(p1, eidx, za1, W1, _agg1_body)
    p2, r2 = _mid(agg1, r1, b1t, Wl2, Wr2, 2000)
    agg2 = _edge_agg(p2, eidx, za2, HID, _agg2_body)
    out = _head(agg2, agg1, r2, b2t, Wfc, bfct, 1000)
    return out


# speedup vs baseline: 1.0069x; 1.0048x over previous
import jax
import jax.numpy as jnp
from jax import lax
from jax.experimental import pallas as pl
from jax.experimental.pallas import tpu as pltpu
from jax.experimental.pallas import tpu_sc as plsc

N = 10000
E = 320000
INC = 128
HID = 32
NHH = 4096

NC, NS, L = 2, 16, 16
NW = NC * NS
CH = 128
NCHUNK = E // CH
CW_Q, CW_R = divmod(NCHUNK, NW)
CPW = 80
NBUF = 8
KAHEAD = 6
N_PAD = 10112
RPT = N_PAD // NS
W1 = HID + L


def _proj_body(x_ref, wl_ref, wr_ref, p_ref, r_ref):
    x = x_ref[...]
    ones = jnp.ones((x.shape[0], L), jnp.float32)
    p_ref[...] = jnp.concatenate(
        [jnp.dot(x, wl_ref[...], preferred_element_type=jnp.float32), ones],
        axis=1)
    r_ref[...] = jnp.dot(x, wr_ref[...], preferred_element_type=jnp.float32)


def _proj(x, Wl, Wr, br):
    n, d = x.shape
    return pl.pallas_call(
        _proj_body,
        grid=(n // br,),
        in_specs=[
            pl.BlockSpec((br, d), lambda i: (i, 0)),
            pl.BlockSpec(Wl.shape, lambda i: (0, 0)),
            pl.BlockSpec(Wr.shape, lambda i: (0, 0)),
        ],
        out_specs=[
            pl.BlockSpec((br, W1), lambda i: (i, 0)),
            pl.BlockSpec((br, HID), lambda i: (i, 0)),
        ],
        out_shape=[jax.ShapeDtypeStruct((n, W1), jnp.float32),
                   jax.ShapeDtypeStruct((n, HID), jnp.float32)],
    )(x, Wl, Wr)


def _mid_body(a_ref, r_ref, b_ref, wl_ref, wr_ref, p_ref, r2_ref):
    a0 = a_ref[0]
    a1 = a_ref[1]
    deg = jnp.maximum(a0[:, HID:HID + 1] + a1[:, HID:HID + 1], 1.0)
    h = jnp.maximum(
        (a0[:, :HID] + a1[:, :HID]) / deg + r_ref[...] + b_ref[0:1, :], 0.0)
    p_ref[...] = jnp.dot(h, wl_ref[...], preferred_element_type=jnp.float32)
    r2_ref[...] = jnp.dot(h, wr_ref[...], preferred_element_type=jnp.float32)


def _mid(agg1, r1, b1, Wl2, Wr2, br):
    n = r1.shape[0]
    row = lambda i: (i, 0)
    whole = lambda i: (0, 0)
    return pl.pallas_call(
        _mid_body,
        grid=(n // br,),
        in_specs=[
            pl.BlockSpec((NC, br, W1), lambda i: (0, i, 0)),
            pl.BlockSpec((br, HID), row),
            pl.BlockSpec((8, HID), whole),
            pl.BlockSpec((HID, HID), whole),
            pl.BlockSpec((HID, HID), whole),
        ],
        out_specs=[
            pl.BlockSpec((br, HID), row),
            pl.BlockSpec((br, HID), row),
        ],
        out_shape=[jax.ShapeDtypeStruct((n, HID), jnp.float32)] * 2,
    )(agg1, r1, b1, Wl2, Wr2)


def _head_body(a_ref, da_ref, r_ref, b_ref, wfc_ref, bfc_ref, o_ref):
    deg = jnp.maximum(
        da_ref[0, :, HID:HID + 1] + da_ref[1, :, HID:HID + 1], 1.0)
    h = jnp.maximum(
        (a_ref[0] + a_ref[1]) / deg + r_ref[...] + b_ref[0:1, :], 0.0)
    o_ref[...] = (jnp.dot(h, wfc_ref[...], preferred_element_type=jnp.float32)
                  + bfc_ref[0:1, :])


def _head(agg2, agg1, r2, b2, Wfc, bfc, br):
    n = r2.shape[0]
    row = lambda i: (i, 0)
    whole = lambda i: (0, 0)
    return pl.pallas_call(
        _head_body,
        grid=(n // br,),
        in_specs=[
            pl.BlockSpec((NC, br, HID), lambda i: (0, i, 0)),
            pl.BlockSpec((NC, br, W1), lambda i: (0, i, 0)),
            pl.BlockSpec((br, HID), row),
            pl.BlockSpec((8, HID), whole),
            pl.BlockSpec((HID, NHH), whole),
            pl.BlockSpec((8, NHH), whole),
        ],
        out_specs=pl.BlockSpec((br, NHH), row),
        out_shape=jax.ShapeDtypeStruct((n, NHH), jnp.float32),
    )(agg2, agg1, r2, b2, Wfc, bfc)


def _agg_impl(width, p_hbm, eidx_hbm, za_hbm, agg_out,
              src_v, dst_v, rows_v, agg_sh, tab_sh, gsems, ssems):
    c = lax.axis_index("c")
    s = lax.axis_index("s")
    wid = s * NC + c
    cw = CW_Q + jnp.where(wid < CW_R, 1, 0)
    start = CW_Q * wid + jnp.minimum(wid, CW_R)

    pltpu.sync_copy(za_hbm.at[pl.ds(s * RPT, RPT)],
                    agg_sh.at[pl.ds(s * RPT, RPT)])

    tpr = N // NS
    pltpu.sync_copy(p_hbm.at[pl.ds(s * tpr, tpr)],
                    tab_sh.at[pl.ds(s * tpr, tpr)])
    pltpu.sync_copy(eidx_hbm.at[0, pl.ds(start, CW_Q)],
                    src_v.at[pl.ds(0, CW_Q)])
    pltpu.sync_copy(eidx_hbm.at[1, pl.ds(start, CW_Q)],
                    dst_v.at[pl.ds(0, CW_Q)])

    @pl.when(cw > CW_Q)
    def _tail():
        pltpu.sync_copy(eidx_hbm.at[0, pl.ds(start + CW_Q, 1)],
                        src_v.at[pl.ds(CW_Q, 1)])
        pltpu.sync_copy(eidx_hbm.at[1, pl.ds(start + CW_Q, 1)],
                        dst_v.at[pl.ds(CW_Q, 1)])

    plsc.subcore_barrier()

    for bb in range(KAHEAD):
        pltpu.async_copy(tab_sh.at[src_v.at[bb]], rows_v.at[bb], gsems[bb])

    def outer(g, carry):
        for bb in range(NBUF):
            j = g * NBUF + bb
            bn = (bb + KAHEAD) % NBUF

            @pl.when((j + KAHEAD >= NBUF) & (j + KAHEAD < CPW)
                     & (j + KAHEAD - NBUF < cw))
            def _free():
                pltpu.make_async_copy(
                    rows_v.at[bn],
                    agg_sh.at[dst_v.at[j + KAHEAD - NBUF]],
                    ssems[bn]).wait()

            @pl.when(j + KAHEAD < cw)
            def _prefetch():
                pltpu.async_copy(
                    tab_sh.at[src_v.at[j + KAHEAD]], rows_v.at[bn], gsems[bn])

            @pl.when(j < cw)
            def _work():
                pltpu.make_async_copy(
                    tab_sh.at[src_v.at[j]], rows_v.at[bb], gsems[bb]).wait()
                pltpu.async_copy(
                    rows_v.at[bb], agg_sh.at[dst_v.at[j]], ssems[bb], add=True)
        return carry

    lax.fori_loop(0, CPW // NBUF, outer, 0)
    for bb in range(NBUF):
        j = CPW - NBUF + bb

        @pl.when(j < cw)
        def _drain():
            pltpu.make_async_copy(
                rows_v.at[bb], agg_sh.at[dst_v.at[j]], ssems[bb]).wait()

    plsc.subcore_barrier()
    pltpu.sync_copy(agg_sh.at[pl.ds(s * RPT, RPT)],
                    agg_out.at[c, pl.ds(s * RPT, RPT)])


def _agg1_body(p_hbm, eidx_hbm, za_hbm, agg_out,
               src_v, dst_v, rows_v, agg_sh, tab_sh, *sems):
    _agg_impl(W1, p_hbm, eidx_hbm, za_hbm, agg_out,
              src_v, dst_v, rows_v, agg_sh, tab_sh, sems[:NBUF], sems[NBUF:])


def _agg2_body(p_hbm, eidx_hbm, za_hbm, agg_out,
               src_v, dst_v, rows_v, agg_sh, tab_sh, *sems):
    _agg_impl(HID, p_hbm, eidx_hbm, za_hbm, agg_out,
              src_v, dst_v, rows_v, agg_sh, tab_sh, sems[:NBUF], sems[NBUF:])


def _edge_agg(p, eidx, za, width, body):
    mesh = plsc.VectorSubcoreMesh(core_axis_name="c", subcore_axis_name="s")
    f = pl.kernel(
        body,
        out_type=jax.ShapeDtypeStruct((NC, N_PAD, width), jnp.float32),
        mesh=mesh,
        scratch_types=[
            pltpu.VMEM((CPW, CH), jnp.int32),
            pltpu.VMEM((CPW, CH), jnp.int32),
            pltpu.VMEM((NBUF, CH, width), jnp.float32),
            pltpu.VMEM_SHARED((N_PAD, width), jnp.float32),
            pltpu.VMEM_SHARED((N, width), jnp.float32),
        ] + [pltpu.SemaphoreType.DMA] * (2 * NBUF),
        compiler_params=pltpu.CompilerParams(use_tc_tiling_on_sc=False),
    )
    return f(p, eidx, za)


def kernel(x, edge_index, Wl1, Wr1, b1, Wl2, Wr2, b2, Wfc, bfc):
    eidx = edge_index.astype(jnp.int32).reshape(2, NCHUNK, CH)
    za1 = jnp.zeros((N_PAD, W1), jnp.float32)
    za2 = jnp.zeros((N_PAD, HID), jnp.float32)
    b1t = jnp.broadcast_to(b1[None, :], (8, HID))
    b2t = jnp.broadcast_to(b2[None, :], (8, HID))
    bfct = jnp.broadcast_to(bfc[None, :], (8, NHH))

    p1, r1 = _proj(x, Wl1, Wr1, N)
    agg1 = _edge_agg(p1, eidx, za1, W1, _agg1_body)
    p2, r2 = _mid(agg1, r1, b1t, Wl2, Wr2, N)
    agg2 = _edge_agg(p2, eidx, za2, HID, _agg2_body)
    out = _head(agg2, agg1, r2, b2t, Wfc, bfct, 1000)
    return out


# speedup vs baseline: 1.0897x; 1.0822x over previous
import jax
import jax.numpy as jnp
from jax import lax
from jax.experimental import pallas as pl
from jax.experimental.pallas import tpu as pltpu
from jax.experimental.pallas import tpu_sc as plsc

N = 10000
E = 320000
INC = 128
HID = 32
NHH = 4096

NC, NS, L = 2, 16, 16
NW = NC * NS
CH = 128
NCHUNK = E // CH
CW_Q, CW_R = divmod(NCHUNK, NW)
CPW = 80
NBUF = 8
KAHEAD = 6
N_PAD = 10112
RPT = N_PAD // NS
W1 = HID + L


def _proj_body(x_ref, wl_ref, wr_ref, p_ref, r_ref):
    x = x_ref[...]
    ones = jnp.ones((x.shape[0], L), jnp.float32)
    p_ref[...] = jnp.concatenate(
        [jnp.dot(x, wl_ref[...], preferred_element_type=jnp.float32), ones],
        axis=1)
    r_ref[...] = jnp.dot(x, wr_ref[...], preferred_element_type=jnp.float32)


def _proj(x, Wl, Wr, br):
    n, d = x.shape
    return pl.pallas_call(
        _proj_body,
        grid=(n // br,),
        in_specs=[
            pl.BlockSpec((br, d), lambda i: (i, 0)),
            pl.BlockSpec(Wl.shape, lambda i: (0, 0)),
            pl.BlockSpec(Wr.shape, lambda i: (0, 0)),
        ],
        out_specs=[
            pl.BlockSpec((br, W1), lambda i: (i, 0)),
            pl.BlockSpec((br, HID), lambda i: (i, 0)),
        ],
        out_shape=[jax.ShapeDtypeStruct((n, W1), jnp.float32),
                   jax.ShapeDtypeStruct((n, HID), jnp.float32)],
    )(x, Wl, Wr)


def _mid_body(a_ref, r_ref, b_ref, wl_ref, wr_ref, p_ref, r2_ref):
    a0 = a_ref[0]
    a1 = a_ref[1]
    deg = jnp.maximum(a0[:, HID:HID + 1] + a1[:, HID:HID + 1], 1.0)
    h = jnp.maximum(
        (a0[:, :HID] + a1[:, :HID]) / deg + r_ref[...] + b_ref[0:1, :], 0.0)
    p_ref[...] = jnp.dot(
        h, wl_ref[...], preferred_element_type=jnp.float32).astype(jnp.bfloat16)
    r2_ref[...] = jnp.dot(h, wr_ref[...], preferred_element_type=jnp.float32)


def _mid(agg1, r1, b1, Wl2, Wr2, br):
    n = r1.shape[0]
    row = lambda i: (i, 0)
    whole = lambda i: (0, 0)
    return pl.pallas_call(
        _mid_body,
        grid=(n // br,),
        in_specs=[
            pl.BlockSpec((NC, br, W1), lambda i: (0, i, 0)),
            pl.BlockSpec((br, HID), row),
            pl.BlockSpec((8, HID), whole),
            pl.BlockSpec((HID, HID), whole),
            pl.BlockSpec((HID, HID), whole),
        ],
        out_specs=[
            pl.BlockSpec((br, HID), row),
            pl.BlockSpec((br, HID), row),
        ],
        out_shape=[jax.ShapeDtypeStruct((n, HID), jnp.bfloat16),
                   jax.ShapeDtypeStruct((n, HID), jnp.float32)],
    )(agg1, r1, b1, Wl2, Wr2)


def _head_body(a_ref, da_ref, r_ref, b_ref, wfc_ref, bfc_ref, o_ref):
    deg = jnp.maximum(
        da_ref[0, :, HID:HID + 1] + da_ref[1, :, HID:HID + 1], 1.0)
    h = jnp.maximum(
        (a_ref[0].astype(jnp.float32) + a_ref[1].astype(jnp.float32)) / deg
        + r_ref[...] + b_ref[0:1, :], 0.0)
    o_ref[...] = (jnp.dot(h, wfc_ref[...], preferred_element_type=jnp.float32)
                  + bfc_ref[0:1, :])


def _head(agg2, agg1, r2, b2, Wfc, bfc, br):
    n = r2.shape[0]
    row = lambda i: (i, 0)
    whole = lambda i: (0, 0)
    return pl.pallas_call(
        _head_body,
        grid=(n // br,),
        in_specs=[
            pl.BlockSpec((NC, br, HID), lambda i: (0, i, 0)),
            pl.BlockSpec((NC, br, W1), lambda i: (0, i, 0)),
            pl.BlockSpec((br, HID), row),
            pl.BlockSpec((8, HID), whole),
            pl.BlockSpec((HID, NHH), whole),
            pl.BlockSpec((8, NHH), whole),
        ],
        out_specs=pl.BlockSpec((br, NHH), row),
        out_shape=jax.ShapeDtypeStruct((n, NHH), jnp.float32),
    )(agg2, agg1, r2, b2, Wfc, bfc)


def _agg_impl(width, p_hbm, eidx_hbm, za_hbm, agg_out,
              src_v, dst_v, rows_v, agg_sh, tab_sh, gsems, ssems):
    c = lax.axis_index("c")
    s = lax.axis_index("s")
    wid = s * NC + c
    cw = CW_Q + jnp.where(wid < CW_R, 1, 0)
    start = CW_Q * wid + jnp.minimum(wid, CW_R)

    pltpu.sync_copy(za_hbm.at[pl.ds(s * RPT, RPT)],
                    agg_sh.at[pl.ds(s * RPT, RPT)])

    tpr = N // NS
    pltpu.sync_copy(p_hbm.at[pl.ds(s * tpr, tpr)],
                    tab_sh.at[pl.ds(s * tpr, tpr)])
    pltpu.sync_copy(eidx_hbm.at[0, pl.ds(start, CW_Q)],
                    src_v.at[pl.ds(0, CW_Q)])
    pltpu.sync_copy(eidx_hbm.at[1, pl.ds(start, CW_Q)],
                    dst_v.at[pl.ds(0, CW_Q)])

    @pl.when(cw > CW_Q)
    def _tail():
        pltpu.sync_copy(eidx_hbm.at[0, pl.ds(start + CW_Q, 1)],
                        src_v.at[pl.ds(CW_Q, 1)])
        pltpu.sync_copy(eidx_hbm.at[1, pl.ds(start + CW_Q, 1)],
                        dst_v.at[pl.ds(CW_Q, 1)])

    plsc.subcore_barrier()

    for bb in range(KAHEAD):
        pltpu.async_copy(tab_sh.at[src_v.at[bb]], rows_v.at[bb], gsems[bb])

    def outer(g, carry):
        for bb in range(NBUF):
            j = g * NBUF + bb
            bn = (bb + KAHEAD) % NBUF

            @pl.when((j + KAHEAD >= NBUF) & (j + KAHEAD < CPW)
                     & (j + KAHEAD - NBUF < cw))
            def _free():
                pltpu.make_async_copy(
                    rows_v.at[bn],
                    agg_sh.at[dst_v.at[j + KAHEAD - NBUF]],
                    ssems[bn]).wait()

            @pl.when(j + KAHEAD < cw)
            def _prefetch():
                pltpu.async_copy(
                    tab_sh.at[src_v.at[j + KAHEAD]], rows_v.at[bn], gsems[bn])

            @pl.when(j < cw)
            def _work():
                pltpu.make_async_copy(
                    tab_sh.at[src_v.at[j]], rows_v.at[bb], gsems[bb]).wait()
                pltpu.async_copy(
                    rows_v.at[bb], agg_sh.at[dst_v.at[j]], ssems[bb], add=True)
        return carry

    lax.fori_loop(0, CPW // NBUF, outer, 0)
    for bb in range(NBUF):
        j = CPW - NBUF + bb

        @pl.when(j < cw)
        def _drain():
            pltpu.make_async_copy(
                rows_v.at[bb], agg_sh.at[dst_v.at[j]], ssems[bb]).wait()

    plsc.subcore_barrier()
    pltpu.sync_copy(agg_sh.at[pl.ds(s * RPT, RPT)],
                    agg_out.at[c, pl.ds(s * RPT, RPT)])


def _agg1_body(p_hbm, eidx_hbm, za_hbm, agg_out,
               src_v, dst_v, rows_v, agg_sh, tab_sh, *sems):
    _agg_impl(W1, p_hbm, eidx_hbm, za_hbm, agg_out,
              src_v, dst_v, rows_v, agg_sh, tab_sh, sems[:NBUF], sems[NBUF:])


def _agg2_body(p_hbm, eidx_hbm, za_hbm, agg_out,
               src_v, dst_v, rows_v, agg_sh, tab_sh, *sems):
    _agg_impl(HID, p_hbm, eidx_hbm, za_hbm, agg_out,
              src_v, dst_v, rows_v, agg_sh, tab_sh, sems[:NBUF], sems[NBUF:])


def _edge_agg(p, eidx, za, width, body, dtype=jnp.float32):
    mesh = plsc.VectorSubcoreMesh(core_axis_name="c", subcore_axis_name="s")
    f = pl.kernel(
        body,
        out_type=jax.ShapeDtypeStruct((NC, N_PAD, width), dtype),
        mesh=mesh,
        scratch_types=[
            pltpu.VMEM((CPW, CH), jnp.int32),
            pltpu.VMEM((CPW, CH), jnp.int32),
            pltpu.VMEM((NBUF, CH, width), dtype),
            pltpu.VMEM_SHARED((N_PAD, width), dtype),
            pltpu.VMEM_SHARED((N, width), dtype),
        ] + [pltpu.SemaphoreType.DMA] * (2 * NBUF),
        compiler_params=pltpu.CompilerParams(use_tc_tiling_on_sc=False),
    )
    return f(p, eidx, za)


def kernel(x, edge_index, Wl1, Wr1, b1, Wl2, Wr2, b2, Wfc, bfc):
    eidx = edge_index.astype(jnp.int32).reshape(2, NCHUNK, CH)
    za1 = jnp.zeros((N_PAD, W1), jnp.float32)
    za2 = jnp.zeros((N_PAD, HID), jnp.bfloat16)
    b1t = jnp.broadcast_to(b1[None, :], (8, HID))
    b2t = jnp.broadcast_to(b2[None, :], (8, HID))
    bfct = jnp.broadcast_to(bfc[None, :], (8, NHH))

    p1, r1 = _proj(x, Wl1, Wr1, N)
    agg1 = _edge_agg(p1, eidx, za1, W1, _agg1_body)
    p2, r2 = _mid(agg1, r1, b1t, Wl2, Wr2, N)
    agg2 = _edge_agg(p2, eidx, za2, HID, _agg2_body, jnp.bfloat16)
    out = _head(agg2, agg1, r2, b2t, Wfc, bfct, 1000)
    return out
